# baseline (speedup 1.0000x reference)
import functools

import jax
import jax.numpy as jnp
import numpy as np
from jax import lax
from jax.experimental import pallas as pl
from jax.experimental.pallas import tpu as pltpu
from jax.experimental.pallas import tpu_sc as plsc
from jax.experimental import layout as jex_layout

B = 4096
EMB = 128
VOCAB = 22
PAD_IDX = 0
PEP_PAD = 3
PEP_LEN = 27
MHC_LEN = 34

NC = 2
NS = 16
NW = NC * NS

PEP_PAD_LEN = 32
MHC_PAD_LEN = 40
BPT = B // NW
PEP_PAD_PER_TILE = BPT * PEP_PAD_LEN
MHC_PAD_PER_TILE = BPT * MHC_PAD_LEN


def _pos_enc(length, emb):
    position = np.arange(length).reshape(-1, 1).astype(np.float32)
    div_term = np.exp(
        np.arange(0, emb, 2).astype(np.float32) * -(np.log(10000.0) / emb))
    pe = np.zeros((length, emb), dtype=np.float32)
    pe[:, 0::2] = np.sin(position * div_term)
    pe[:, 1::2] = np.cos(position * div_term)
    return pe


def _pep_posext():
    pe = np.zeros((PEP_LEN, EMB), dtype=np.float32)
    pe[PEP_PAD:PEP_PAD + 21] = _pos_enc(21, EMB)
    return pe


_PEP_POSEXT = _pep_posext()
_MHC_POS = _pos_enc(MHC_LEN, EMB)


def _prep_body(pw_ref, mw_ref, px_ref, mx_ref, pe_ref, me_ref,
               tpep_ref, tmhc_ref, mask_ref, ppad_ref, mpad_ref):
    vnz = (lax.broadcasted_iota(jnp.int32, (VOCAB, EMB), 0) != PAD_IDX)
    vnz = vnz.astype(jnp.float32)
    tpep_ref[...] = (pw_ref[...][None, :, :]
                     + pe_ref[...][:, None, :] * vnz[None, :, :])
    tmhc_ref[...] = mw_ref[...][None, :, :] + me_ref[...][:, None, :]
    mask_ref[...] = px_ref[:, PEP_PAD:PEP_PAD + 21] != PAD_IDX
    ppad_ref[:, :PEP_LEN] = px_ref[...]
    ppad_ref[:, PEP_LEN:] = jnp.zeros((B, PEP_PAD_LEN - PEP_LEN), jnp.int32)
    mpad_ref[:, :MHC_LEN] = mx_ref[...]
    mpad_ref[:, MHC_LEN:] = jnp.zeros((B, MHC_PAD_LEN - MHC_LEN), jnp.int32)


_prep = pl.pallas_call(
    _prep_body,
    out_shape=(
        jax.ShapeDtypeStruct((PEP_LEN, VOCAB, EMB), jnp.float32),
        jax.ShapeDtypeStruct((MHC_LEN, VOCAB, EMB), jnp.float32),
        jax.ShapeDtypeStruct((B, 21), jnp.bool_),
        jax.ShapeDtypeStruct((B, PEP_PAD_LEN), jnp.int32),
        jax.ShapeDtypeStruct((B, MHC_PAD_LEN), jnp.int32),
    ),
)


def _sc_body(tpep_hbm, tmhc_hbm, pxp_hbm, mxp_hbm,
             pout_hbm, mout_hbm, x_v, idx_v, rows_v,
             sga, sgb, soa, sob):
    wid = lax.axis_index("s") * NC + lax.axis_index("c")
    lane = lax.iota(jnp.int32, 16)

    def run_table(xp_hbm, tbl_hbm, out_hbm, npad, pad_len, seq_len, off_g):
        base = wid * npad
        pltpu.sync_copy(xp_hbm.at[pl.ds(base, npad)], x_v.at[pl.ds(0, npad)])
        ng = len(off_g)

        def idx_block(q, carry):
            d = q * 16 * ng
            for g in range(ng):
                s = pl.ds(d + 16 * g, 16)
                idx_v[s] = x_v[s] + off_g[g]
            return carry

        lax.fori_loop(0, npad // (16 * ng), idx_block, 0)

        buf_a = rows_v.at[0, pl.ds(0, seq_len)]
        buf_b = rows_v.at[1, pl.ds(0, seq_len)]
        obase = wid * BPT

        def gstart(b, buf, sem):
            pltpu.async_copy(
                tbl_hbm.at[idx_v.at[pl.ds(b * pad_len, seq_len)]], buf, sem)

        def gwait(buf, sem):
            pltpu.make_async_copy(
                tbl_hbm.at[idx_v.at[pl.ds(0, seq_len)]], buf, sem).wait()

        def ostart(b, buf, sem):
            pltpu.async_copy(buf, out_hbm.at[obase + b], sem)

        def owait(buf, sem):
            pltpu.make_async_copy(buf, out_hbm.at[obase], sem).wait()

        gstart(0, buf_a, sga)
        gstart(1, buf_b, sgb)

        def body(i, carry):
            bb = 2 * i
            gwait(buf_a, sga)
            ostart(bb, buf_a, soa)
            gwait(buf_b, sgb)
            ostart(bb + 1, buf_b, sob)

            @pl.when(i < BPT // 2 - 1)
            def _():
                owait(buf_a, soa)
                gstart(bb + 2, buf_a, sga)
                owait(buf_b, sob)
                gstart(bb + 3, buf_b, sgb)

            return carry

        lax.fori_loop(0, BPT // 2, body, 0)
        owait(buf_a, soa)
        owait(buf_b, sob)

    pep_off = [
        lane * VOCAB,
        jnp.where(lane < 11, (16 + lane) * VOCAB, 26 * VOCAB),
    ]
    mhc_off = [
        lane * VOCAB,
        (16 + lane) * VOCAB,
        jnp.where(lane < 2, (32 + lane) * VOCAB,
                  jnp.where(lane >= 8, (lane - 8) * VOCAB, 33 * VOCAB)),
        (8 + lane) * VOCAB,
        jnp.where(lane < 10, (24 + lane) * VOCAB, 33 * VOCAB),
    ]
    run_table(pxp_hbm, tpep_hbm, pout_hbm,
              PEP_PAD_PER_TILE, PEP_PAD_LEN, PEP_LEN, pep_off)
    run_table(mxp_hbm, tmhc_hbm, mout_hbm,
              MHC_PAD_PER_TILE, MHC_PAD_LEN, MHC_LEN, mhc_off)


@functools.cache
def _make_gather():
    return pl.kernel(
        _sc_body,
        out_type=(
            jax.ShapeDtypeStruct((B, PEP_LEN, EMB), jnp.float32),
            jax.ShapeDtypeStruct((B, MHC_LEN, EMB), jnp.float32),
        ),
        mesh=plsc.VectorSubcoreMesh(core_axis_name="c", subcore_axis_name="s"),
        compiler_params=pltpu.CompilerParams(use_tc_tiling_on_sc=True),
        scratch_types=[
            pltpu.VMEM((MHC_PAD_PER_TILE,), jnp.int32),
            pltpu.VMEM((MHC_PAD_PER_TILE,), jnp.int32),
            pltpu.VMEM((2, MHC_LEN, EMB), jnp.float32),
            pltpu.SemaphoreType.DMA,
            pltpu.SemaphoreType.DMA,
            pltpu.SemaphoreType.DMA,
            pltpu.SemaphoreType.DMA,
        ],
    )


def _kernel_impl(peptide_x, mhc_x, peptide_W, mhc_W):
    px = peptide_x.astype(jnp.int32)
    mx = mhc_x.astype(jnp.int32)
    tpep3, tmhc3, masks, ppad, mpad = _prep(
        peptide_W, mhc_W, px, mx,
        jnp.asarray(_PEP_POSEXT), jnp.asarray(_MHC_POS))
    tpep = tpep3.reshape(PEP_LEN * VOCAB, EMB)
    tmhc = tmhc3.reshape(MHC_LEN * VOCAB, EMB)
    pout3, mout3 = _make_gather()(tpep, tmhc,
                                  ppad.reshape(-1), mpad.reshape(-1))
    return (pout3, mout3, masks)


@functools.cache
def _jitted_kernel():
    sharding = jax.sharding.SingleDeviceSharding(jax.devices()[0])
    fmt = jex_layout.Format(
        jex_layout.Layout(major_to_minor=(0, 1, 2), tiling=((8, 128),)),
        sharding)
    return jax.jit(_kernel_impl, out_shardings=(fmt, fmt, None))


def kernel(peptide_x, mhc_x, peptide_W, mhc_W):
    return _jitted_kernel()(peptide_x, mhc_x, peptide_W, mhc_W)


# speedup vs baseline: 1.0795x; 1.0795x over previous
import functools

import jax
import jax.numpy as jnp
import numpy as np
from jax import lax
from jax.experimental import pallas as pl
from jax.experimental.pallas import tpu as pltpu
from jax.experimental.pallas import tpu_sc as plsc
from jax.experimental import layout as jex_layout

B = 4096
EMB = 128
VOCAB = 22
PAD_IDX = 0
PEP_PAD = 3
PEP_LEN = 27
MHC_LEN = 34

NC = 2
NS = 16
NW = NC * NS

PEP_PAD_LEN = 32
MHC_PAD_LEN = 40
BPT = B // NW
PEP_PAD_PER_TILE = BPT * PEP_PAD_LEN
MHC_PAD_PER_TILE = BPT * MHC_PAD_LEN


def _pos_enc(length, emb):
    position = np.arange(length).reshape(-1, 1).astype(np.float32)
    div_term = np.exp(
        np.arange(0, emb, 2).astype(np.float32) * -(np.log(10000.0) / emb))
    pe = np.zeros((length, emb), dtype=np.float32)
    pe[:, 0::2] = np.sin(position * div_term)
    pe[:, 1::2] = np.cos(position * div_term)
    return pe


def _pep_posext():
    pe = np.zeros((PEP_LEN, EMB), dtype=np.float32)
    pe[PEP_PAD:PEP_PAD + 21] = _pos_enc(21, EMB)
    return pe


_PEP_POSEXT = _pep_posext()
_MHC_POS = _pos_enc(MHC_LEN, EMB)


def _prep_body(pw_ref, mw_ref, px_ref, mx_ref, pe_ref, me_ref,
               tpep_ref, tmhc_ref, mask_ref, ppad_ref, mpad_ref):
    vnz = (lax.broadcasted_iota(jnp.int32, (VOCAB, EMB), 0) != PAD_IDX)
    vnz = vnz.astype(jnp.float32)
    tpep_ref[...] = (pw_ref[...][None, :, :]
                     + pe_ref[...][:, None, :] * vnz[None, :, :])
    tmhc_ref[...] = mw_ref[...][None, :, :] + me_ref[...][:, None, :]
    mask_ref[...] = px_ref[:, PEP_PAD:PEP_PAD + 21] != PAD_IDX
    ppad_ref[:, :PEP_LEN] = px_ref[...]
    ppad_ref[:, PEP_LEN:] = jnp.zeros((B, PEP_PAD_LEN - PEP_LEN), jnp.int32)
    mpad_ref[:, :MHC_LEN] = mx_ref[...]
    mpad_ref[:, MHC_LEN:] = jnp.zeros((B, MHC_PAD_LEN - MHC_LEN), jnp.int32)


_prep = pl.pallas_call(
    _prep_body,
    out_shape=(
        jax.ShapeDtypeStruct((PEP_LEN, VOCAB, EMB), jnp.float32),
        jax.ShapeDtypeStruct((MHC_LEN, VOCAB, EMB), jnp.float32),
        jax.ShapeDtypeStruct((B, 21), jnp.bool_),
        jax.ShapeDtypeStruct((B, PEP_PAD_LEN), jnp.int32),
        jax.ShapeDtypeStruct((B, MHC_PAD_LEN), jnp.int32),
    ),
)


_NBUF = 4


def _sc_body(tpep_hbm, tmhc_hbm, pxp_hbm, mxp_hbm,
             pout_hbm, mout_hbm, x_v, idx_v, rows_v,
             sg0, sg1, sg2, sg3, so0, so1, so2, so3):
    wid = lax.axis_index("s") * NC + lax.axis_index("c")
    lane = lax.iota(jnp.int32, 16)
    sg = (sg0, sg1, sg2, sg3)
    so = (so0, so1, so2, so3)

    def run_table(xp_hbm, tbl_hbm, out_hbm, npad, pad_len, seq_len, off_g):
        base = wid * npad
        pltpu.sync_copy(xp_hbm.at[pl.ds(base, npad)], x_v.at[pl.ds(0, npad)])
        ng = len(off_g)

        def idx_block(q, carry):
            d = q * 16 * ng
            for g in range(ng):
                s = pl.ds(d + 16 * g, 16)
                idx_v[s] = x_v[s] + off_g[g]
            return carry

        lax.fori_loop(0, npad // (16 * ng), idx_block, 0)

        bufs = [rows_v.at[k, pl.ds(0, seq_len)] for k in range(_NBUF)]
        obase = wid * BPT

        def gstart(b, buf, sem):
            pltpu.async_copy(
                tbl_hbm.at[idx_v.at[pl.ds(b * pad_len, seq_len)]], buf, sem)

        def gwait(buf, sem):
            pltpu.make_async_copy(
                tbl_hbm.at[idx_v.at[pl.ds(0, seq_len)]], buf, sem).wait()

        def ostart(b, buf, sem):
            pltpu.async_copy(buf, out_hbm.at[obase + b], sem)

        def owait(buf, sem):
            pltpu.make_async_copy(buf, out_hbm.at[obase], sem).wait()

        for k in range(_NBUF):
            gstart(k, bufs[k], sg[k])

        def body(i, carry):
            bb = _NBUF * i
            for k in range(_NBUF):
                gwait(bufs[k], sg[k])
                ostart(bb + k, bufs[k], so[k])

            @pl.when(i < BPT // _NBUF - 1)
            def _():
                for k in range(_NBUF):
                    owait(bufs[k], so[k])
                    gstart(bb + _NBUF + k, bufs[k], sg[k])

            return carry

        lax.fori_loop(0, BPT // _NBUF, body, 0)
        for k in range(_NBUF):
            owait(bufs[k], so[k])

    pep_off = [
        lane * VOCAB,
        jnp.where(lane < 11, (16 + lane) * VOCAB, 26 * VOCAB),
    ]
    mhc_off = [
        lane * VOCAB,
        (16 + lane) * VOCAB,
        jnp.where(lane < 2, (32 + lane) * VOCAB,
                  jnp.where(lane >= 8, (lane - 8) * VOCAB, 33 * VOCAB)),
        (8 + lane) * VOCAB,
        jnp.where(lane < 10, (24 + lane) * VOCAB, 33 * VOCAB),
    ]
    run_table(pxp_hbm, tpep_hbm, pout_hbm,
              PEP_PAD_PER_TILE, PEP_PAD_LEN, PEP_LEN, pep_off)
    run_table(mxp_hbm, tmhc_hbm, mout_hbm,
              MHC_PAD_PER_TILE, MHC_PAD_LEN, MHC_LEN, mhc_off)


@functools.cache
def _make_gather():
    return pl.kernel(
        _sc_body,
        out_type=(
            jax.ShapeDtypeStruct((B, PEP_LEN, EMB), jnp.float32),
            jax.ShapeDtypeStruct((B, MHC_LEN, EMB), jnp.float32),
        ),
        mesh=plsc.VectorSubcoreMesh(core_axis_name="c", subcore_axis_name="s"),
        compiler_params=pltpu.CompilerParams(use_tc_tiling_on_sc=True),
        scratch_types=[
            pltpu.VMEM((MHC_PAD_PER_TILE,), jnp.int32),
            pltpu.VMEM((MHC_PAD_PER_TILE,), jnp.int32),
            pltpu.VMEM((_NBUF, MHC_LEN, EMB), jnp.float32),
        ] + [pltpu.SemaphoreType.DMA] * (2 * _NBUF),
    )


def _kernel_impl(peptide_x, mhc_x, peptide_W, mhc_W):
    px = peptide_x.astype(jnp.int32)
    mx = mhc_x.astype(jnp.int32)
    tpep3, tmhc3, masks, ppad, mpad = _prep(
        peptide_W, mhc_W, px, mx,
        jnp.asarray(_PEP_POSEXT), jnp.asarray(_MHC_POS))
    tpep = tpep3.reshape(PEP_LEN * VOCAB, EMB)
    tmhc = tmhc3.reshape(MHC_LEN * VOCAB, EMB)
    pout3, mout3 = _make_gather()(tpep, tmhc,
                                  ppad.reshape(-1), mpad.reshape(-1))
    return (pout3, mout3, masks)


@functools.cache
def _jitted_kernel():
    sharding = jax.sharding.SingleDeviceSharding(jax.devices()[0])
    fmt = jex_layout.Format(
        jex_layout.Layout(major_to_minor=(0, 1, 2), tiling=((8, 128),)),
        sharding)
    return jax.jit(_kernel_impl, out_shardings=(fmt, fmt, None))


def kernel(peptide_x, mhc_x, peptide_W, mhc_W):
    return _jitted_kernel()(peptide_x, mhc_x, peptide_W, mhc_W)
